# Initial kernel scaffold; baseline (speedup 1.0000x reference)
import jax
import jax.numpy as jnp
from jax.experimental import pallas as pl


def kernel(xyz1, xyz2, points1, points2, point_lens, embedding_lens, point_mask, W0, g0, b0, W1, g1, b1):
    raise NotImplementedError("write your pallas kernel here")


# speedup vs baseline: 17.4421x; 17.4421x over previous
import functools

import jax
import jax.numpy as jnp
from jax.experimental import pallas as pl
from jax.experimental.pallas import tpu as pltpu

B, N, S, D1, D2, K = 8, 4096, 1024, 128, 256, 5
C0, C1 = 256, 128
BN = 512
INF = 3e38
EPS = 1.1920928955078125e-07


def _knn_interp_body(plens_ref, elens_ref, xyz1_ref, xyz2t_ref, p1_ref, p2_ref,
                     w0t_ref, h0_ref, stats_ref):
    b = pl.program_id(0)
    nb = pl.program_id(1)

    a = xyz1_ref[0]
    bt = xyz2t_ref[0]
    an = jnp.sum(a * a, axis=1, keepdims=True)
    bn_sq = jnp.sum(bt * bt, axis=0, keepdims=True)
    ab = jax.lax.dot_general(a, bt, (((1,), (0,)), ((), ())),
                             precision=jax.lax.Precision.HIGHEST,
                             preferred_element_type=jnp.float32)
    d2 = jnp.maximum(an - 2.0 * ab + bn_sq, 0.0)

    col = jax.lax.broadcasted_iota(jnp.int32, (BN, S), 1)
    elen = elens_ref[b]
    d = jnp.where(col < elen, d2, INF)

    wsp = jnp.zeros((BN, S), jnp.float32)
    wsum = jnp.zeros((BN, 1), jnp.float32)
    for _ in range(K):
        m = jnp.min(d, axis=1, keepdims=True)
        pick_col = jnp.min(jnp.where(d == m, col, S), axis=1,
                           keepdims=True)
        pick = col == pick_col
        wk = 1.0 / (m + EPS)
        wsp = wsp + jnp.where(pick, wk, 0.0)
        wsum = wsum + wk
        d = jnp.where(pick, INF, d)
    wsp = wsp / wsum

    interp = jax.lax.dot_general(wsp, p2_ref[0], (((1,), (0,)), ((), ())),
                                 preferred_element_type=jnp.float32)
    w0t = w0t_ref[...]
    h0 = (jax.lax.dot_general(p1_ref[0], w0t[:D1], (((1,), (0,)), ((), ())),
                              preferred_element_type=jnp.float32)
          + jax.lax.dot_general(interp, w0t[D1:], (((1,), (0,)), ((), ())),
                                preferred_element_type=jnp.float32))
    h0_ref[0] = h0

    row = jax.lax.broadcasted_iota(jnp.int32, (BN, 1), 0) + nb * BN
    m_row = (row < plens_ref[b]).astype(jnp.float32)
    s1 = jnp.sum(h0 * m_row, axis=0, keepdims=True)
    s2 = jnp.sum(h0 * h0 * m_row, axis=0, keepdims=True)
    riota = jax.lax.broadcasted_iota(jnp.int32, (8, C0), 0)
    contrib = (jnp.where(riota == 0, jnp.broadcast_to(s1, (8, C0)), 0.0)
               + jnp.where(riota == 1, jnp.broadcast_to(s2, (8, C0)), 0.0))

    @pl.when(jnp.logical_and(b == 0, nb == 0))
    def _():
        stats_ref[...] = jnp.zeros((8, C0), jnp.float32)

    stats_ref[...] += contrib


def _n_valid(plens_ref):
    nv = jnp.int32(0)
    for i in range(B):
        nv = nv + plens_ref[i]
    return jnp.maximum(nv.astype(jnp.float32), 1.0)


def _bn_mlp_body(plens_ref, h0_ref, stats_ref, g_ref, bias_ref, w1t_ref,
                 h1_ref, stats2_ref, *, cin, cout):
    b = pl.program_id(0)
    nb = pl.program_id(1)
    nv = _n_valid(plens_ref)
    stats = stats_ref[...]
    mean = stats[0:1, :] / nv
    var = stats[1:2, :] / nv - mean * mean
    scale = g_ref[...] * jax.lax.rsqrt(var + 1e-5)
    shift = bias_ref[...] - mean * scale
    xn = jnp.maximum(h0_ref[0] * scale + shift, 0.0)
    h1 = jax.lax.dot_general(xn, w1t_ref[...], (((1,), (0,)), ((), ())),
                             preferred_element_type=jnp.float32)
    h1_ref[0] = h1

    row = jax.lax.broadcasted_iota(jnp.int32, (BN, 1), 0) + nb * BN
    m_row = (row < plens_ref[b]).astype(jnp.float32)
    s1 = jnp.sum(h1 * m_row, axis=0, keepdims=True)
    s2 = jnp.sum(h1 * h1 * m_row, axis=0, keepdims=True)
    riota = jax.lax.broadcasted_iota(jnp.int32, (8, cout), 0)
    contrib = (jnp.where(riota == 0, jnp.broadcast_to(s1, (8, cout)), 0.0)
               + jnp.where(riota == 1, jnp.broadcast_to(s2, (8, cout)), 0.0))

    @pl.when(jnp.logical_and(b == 0, nb == 0))
    def _():
        stats2_ref[...] = jnp.zeros((8, cout), jnp.float32)

    stats2_ref[...] += contrib


def _bn_final_body(plens_ref, h1_ref, stats_ref, g_ref, bias_ref, out_ref, *,
                   cin):
    nv = _n_valid(plens_ref)
    stats = stats_ref[...]
    mean = stats[0:1, :] / nv
    var = stats[1:2, :] / nv - mean * mean
    scale = g_ref[...] * jax.lax.rsqrt(var + 1e-5)
    shift = bias_ref[...] - mean * scale
    out_ref[0] = jnp.maximum(h1_ref[0] * scale + shift, 0.0)


def kernel(xyz1, xyz2, points1, points2, point_lens, embedding_lens,
           point_mask, W0, g0, b0, W1, g1, b1):
    del point_mask
    xyz2t = xyz2.transpose(0, 2, 1)
    w0t = W0.T
    w1t = W1.T
    g0r, b0r = g0.reshape(1, C0), b0.reshape(1, C0)
    g1r, b1r = g1.reshape(1, C1), b1.reshape(1, C1)

    grid = (B, N // BN)
    smem = pl.BlockSpec(memory_space=pltpu.SMEM)

    h0, stats0 = pl.pallas_call(
        _knn_interp_body,
        grid=grid,
        in_specs=[
            smem, smem,
            pl.BlockSpec((1, BN, 3), lambda b, n: (b, n, 0)),
            pl.BlockSpec((1, 3, S), lambda b, n: (b, 0, 0)),
            pl.BlockSpec((1, BN, D1), lambda b, n: (b, n, 0)),
            pl.BlockSpec((1, S, D2), lambda b, n: (b, 0, 0)),
            pl.BlockSpec((D1 + D2, C0), lambda b, n: (0, 0)),
        ],
        out_specs=[
            pl.BlockSpec((1, BN, C0), lambda b, n: (b, n, 0)),
            pl.BlockSpec((8, C0), lambda b, n: (0, 0)),
        ],
        out_shape=[
            jax.ShapeDtypeStruct((B, N, C0), jnp.float32),
            jax.ShapeDtypeStruct((8, C0), jnp.float32),
        ],
    )(point_lens, embedding_lens, xyz1, xyz2t, points1, points2, w0t)

    h1, stats1 = pl.pallas_call(
        functools.partial(_bn_mlp_body, cin=C0, cout=C1),
        grid=grid,
        in_specs=[
            smem,
            pl.BlockSpec((1, BN, C0), lambda b, n: (b, n, 0)),
            pl.BlockSpec((8, C0), lambda b, n: (0, 0)),
            pl.BlockSpec((1, C0), lambda b, n: (0, 0)),
            pl.BlockSpec((1, C0), lambda b, n: (0, 0)),
            pl.BlockSpec((C0, C1), lambda b, n: (0, 0)),
        ],
        out_specs=[
            pl.BlockSpec((1, BN, C1), lambda b, n: (b, n, 0)),
            pl.BlockSpec((8, C1), lambda b, n: (0, 0)),
        ],
        out_shape=[
            jax.ShapeDtypeStruct((B, N, C1), jnp.float32),
            jax.ShapeDtypeStruct((8, C1), jnp.float32),
        ],
    )(point_lens, h0, stats0, g0r, b0r, w1t)

    out = pl.pallas_call(
        functools.partial(_bn_final_body, cin=C1),
        grid=grid,
        in_specs=[
            smem,
            pl.BlockSpec((1, BN, C1), lambda b, n: (b, n, 0)),
            pl.BlockSpec((8, C1), lambda b, n: (0, 0)),
            pl.BlockSpec((1, C1), lambda b, n: (0, 0)),
            pl.BlockSpec((1, C1), lambda b, n: (0, 0)),
        ],
        out_specs=pl.BlockSpec((1, BN, C1), lambda b, n: (b, n, 0)),
        out_shape=jax.ShapeDtypeStruct((B, N, C1), jnp.float32),
    )(point_lens, h1, stats1, g1r, b1r)

    return out


# speedup vs baseline: 24.7303x; 1.4178x over previous
import functools

import jax
import jax.numpy as jnp
from jax.experimental import pallas as pl
from jax.experimental.pallas import tpu as pltpu

B, N, S, D1, D2, K = 8, 4096, 1024, 128, 256, 5
C0, C1 = 256, 128
BN = 512
INF = 3e38
BIG = 1e37
EPS = 1.1920928955078125e-07


def _knn_interp_body(plens_ref, elens_ref, xyz1_ref, xyz2t_ref, p1_ref, p2_ref,
                     w0t_ref, h0_ref, stats_ref):
    b = pl.program_id(0)
    nb = pl.program_id(1)

    a = xyz1_ref[0]
    bt = xyz2t_ref[0]
    an = jnp.sum(a * a, axis=1, keepdims=True)
    bn_sq = jnp.sum(bt * bt, axis=0, keepdims=True)
    col_s = jax.lax.broadcasted_iota(jnp.int32, (1, S), 1)
    elen = elens_ref[b]
    bn_m = jnp.where(col_s < elen, bn_sq, BIG)
    ab2 = jax.lax.dot_general(a * (-2.0), bt, (((1,), (0,)), ((), ())),
                              precision=jax.lax.Precision.HIGHEST,
                              preferred_element_type=jnp.float32)
    d0 = jnp.maximum(ab2 + an + bn_m, 0.0)

    d = d0
    wsum = jnp.zeros((BN, 1), jnp.float32)
    for _ in range(K):
        m = jnp.min(d, axis=1, keepdims=True)
        wsum = wsum + 1.0 / (m + EPS)
        d = jnp.where(d == m, INF, d)

    wsp = jnp.where(d == INF, 1.0 / (d0 + EPS), 0.0)
    interp = jax.lax.dot_general(wsp, p2_ref[0], (((1,), (0,)), ((), ())),
                                 preferred_element_type=jnp.float32)
    interp = interp * (1.0 / wsum)
    w0t = w0t_ref[...]
    h0 = (jax.lax.dot_general(p1_ref[0], w0t[:D1], (((1,), (0,)), ((), ())),
                              preferred_element_type=jnp.float32)
          + jax.lax.dot_general(interp, w0t[D1:], (((1,), (0,)), ((), ())),
                                preferred_element_type=jnp.float32))
    h0_ref[0] = h0

    row = jax.lax.broadcasted_iota(jnp.int32, (BN, 1), 0) + nb * BN
    m_row = (row < plens_ref[b]).astype(jnp.float32)
    s1 = jnp.sum(h0 * m_row, axis=0, keepdims=True)
    s2 = jnp.sum(h0 * h0 * m_row, axis=0, keepdims=True)
    riota = jax.lax.broadcasted_iota(jnp.int32, (8, C0), 0)
    contrib = (jnp.where(riota == 0, jnp.broadcast_to(s1, (8, C0)), 0.0)
               + jnp.where(riota == 1, jnp.broadcast_to(s2, (8, C0)), 0.0))

    @pl.when(jnp.logical_and(b == 0, nb == 0))
    def _():
        stats_ref[...] = jnp.zeros((8, C0), jnp.float32)

    stats_ref[...] += contrib


def _n_valid(plens_ref):
    nv = jnp.int32(0)
    for i in range(B):
        nv = nv + plens_ref[i]
    return jnp.maximum(nv.astype(jnp.float32), 1.0)


def _bn_mlp_body(plens_ref, h0_ref, stats_ref, g_ref, bias_ref, w1t_ref,
                 h1_ref, stats2_ref, *, cin, cout):
    b = pl.program_id(0)
    nb = pl.program_id(1)
    nv = _n_valid(plens_ref)
    stats = stats_ref[...]
    mean = stats[0:1, :] / nv
    var = stats[1:2, :] / nv - mean * mean
    scale = g_ref[...] * jax.lax.rsqrt(var + 1e-5)
    shift = bias_ref[...] - mean * scale
    xn = jnp.maximum(h0_ref[0] * scale + shift, 0.0)
    h1 = jax.lax.dot_general(xn, w1t_ref[...], (((1,), (0,)), ((), ())),
                             preferred_element_type=jnp.float32)
    h1_ref[0] = h1

    row = jax.lax.broadcasted_iota(jnp.int32, (BN, 1), 0) + nb * BN
    m_row = (row < plens_ref[b]).astype(jnp.float32)
    s1 = jnp.sum(h1 * m_row, axis=0, keepdims=True)
    s2 = jnp.sum(h1 * h1 * m_row, axis=0, keepdims=True)
    riota = jax.lax.broadcasted_iota(jnp.int32, (8, cout), 0)
    contrib = (jnp.where(riota == 0, jnp.broadcast_to(s1, (8, cout)), 0.0)
               + jnp.where(riota == 1, jnp.broadcast_to(s2, (8, cout)), 0.0))

    @pl.when(jnp.logical_and(b == 0, nb == 0))
    def _():
        stats2_ref[...] = jnp.zeros((8, cout), jnp.float32)

    stats2_ref[...] += contrib


def _bn_final_body(plens_ref, h1_ref, stats_ref, g_ref, bias_ref, out_ref, *,
                   cin):
    nv = _n_valid(plens_ref)
    stats = stats_ref[...]
    mean = stats[0:1, :] / nv
    var = stats[1:2, :] / nv - mean * mean
    scale = g_ref[...] * jax.lax.rsqrt(var + 1e-5)
    shift = bias_ref[...] - mean * scale
    out_ref[0] = jnp.maximum(h1_ref[0] * scale + shift, 0.0)


def kernel(xyz1, xyz2, points1, points2, point_lens, embedding_lens,
           point_mask, W0, g0, b0, W1, g1, b1):
    del point_mask
    xyz2t = xyz2.transpose(0, 2, 1)
    w0t = W0.T
    w1t = W1.T
    g0r, b0r = g0.reshape(1, C0), b0.reshape(1, C0)
    g1r, b1r = g1.reshape(1, C1), b1.reshape(1, C1)

    grid = (B, N // BN)
    smem = pl.BlockSpec(memory_space=pltpu.SMEM)

    h0, stats0 = pl.pallas_call(
        _knn_interp_body,
        grid=grid,
        in_specs=[
            smem, smem,
            pl.BlockSpec((1, BN, 3), lambda b, n: (b, n, 0)),
            pl.BlockSpec((1, 3, S), lambda b, n: (b, 0, 0)),
            pl.BlockSpec((1, BN, D1), lambda b, n: (b, n, 0)),
            pl.BlockSpec((1, S, D2), lambda b, n: (b, 0, 0)),
            pl.BlockSpec((D1 + D2, C0), lambda b, n: (0, 0)),
        ],
        out_specs=[
            pl.BlockSpec((1, BN, C0), lambda b, n: (b, n, 0)),
            pl.BlockSpec((8, C0), lambda b, n: (0, 0)),
        ],
        out_shape=[
            jax.ShapeDtypeStruct((B, N, C0), jnp.float32),
            jax.ShapeDtypeStruct((8, C0), jnp.float32),
        ],
    )(point_lens, embedding_lens, xyz1, xyz2t, points1, points2, w0t)

    h1, stats1 = pl.pallas_call(
        functools.partial(_bn_mlp_body, cin=C0, cout=C1),
        grid=grid,
        in_specs=[
            smem,
            pl.BlockSpec((1, BN, C0), lambda b, n: (b, n, 0)),
            pl.BlockSpec((8, C0), lambda b, n: (0, 0)),
            pl.BlockSpec((1, C0), lambda b, n: (0, 0)),
            pl.BlockSpec((1, C0), lambda b, n: (0, 0)),
            pl.BlockSpec((C0, C1), lambda b, n: (0, 0)),
        ],
        out_specs=[
            pl.BlockSpec((1, BN, C1), lambda b, n: (b, n, 0)),
            pl.BlockSpec((8, C1), lambda b, n: (0, 0)),
        ],
        out_shape=[
            jax.ShapeDtypeStruct((B, N, C1), jnp.float32),
            jax.ShapeDtypeStruct((8, C1), jnp.float32),
        ],
    )(point_lens, h0, stats0, g0r, b0r, w1t)

    out = pl.pallas_call(
        functools.partial(_bn_final_body, cin=C1),
        grid=grid,
        in_specs=[
            smem,
            pl.BlockSpec((1, BN, C1), lambda b, n: (b, n, 0)),
            pl.BlockSpec((8, C1), lambda b, n: (0, 0)),
            pl.BlockSpec((1, C1), lambda b, n: (0, 0)),
            pl.BlockSpec((1, C1), lambda b, n: (0, 0)),
        ],
        out_specs=pl.BlockSpec((1, BN, C1), lambda b, n: (b, n, 0)),
        out_shape=jax.ShapeDtypeStruct((B, N, C1), jnp.float32),
    )(point_lens, h1, stats1, g1r, b1r)

    return out


# speedup vs baseline: 28.8538x; 1.1667x over previous
import functools

import jax
import jax.numpy as jnp
from jax.experimental import pallas as pl
from jax.experimental.pallas import tpu as pltpu

B, N, S, D1, D2, K = 8, 4096, 1024, 128, 256, 5
C0, C1 = 256, 128
BN = 1024
INF = 3e38
BIG = 1e37
EPS = 1.1920928955078125e-07


def _knn_interp_body(plens_ref, elens_ref, xyz1_ref, xyz2t_ref, p1_ref, p2_ref,
                     w0t_ref, h0_ref, stats_ref):
    b = pl.program_id(0)
    nb = pl.program_id(1)

    a = xyz1_ref[0]
    bt = xyz2t_ref[0]
    an = jnp.sum(a * a, axis=1, keepdims=True)
    bn_sq = jnp.sum(bt * bt, axis=0, keepdims=True)
    col_s = jax.lax.broadcasted_iota(jnp.int32, (1, S), 1)
    elen = elens_ref[b]
    bn_m = jnp.where(col_s < elen, bn_sq, BIG)
    ab2 = jax.lax.dot_general(a * (-2.0), bt, (((1,), (0,)), ((), ())),
                              precision=jax.lax.Precision.HIGHEST,
                              preferred_element_type=jnp.float32)
    d0 = jnp.maximum(ab2 + an + bn_m, 0.0)

    d = d0
    wsum = jnp.zeros((BN, 1), jnp.float32)
    for _ in range(K):
        m = jnp.min(d, axis=1, keepdims=True)
        wsum = wsum + 1.0 / (m + EPS)
        d = jnp.where(d == m, INF, d)

    wsp = jnp.where(d == INF, 1.0 / (d0 + EPS), 0.0)
    interp = jax.lax.dot_general(wsp, p2_ref[0], (((1,), (0,)), ((), ())),
                                 preferred_element_type=jnp.float32)
    interp = interp * (1.0 / wsum)
    w0t = w0t_ref[...]
    h0 = (jax.lax.dot_general(p1_ref[0], w0t[:D1], (((1,), (0,)), ((), ())),
                              preferred_element_type=jnp.float32)
          + jax.lax.dot_general(interp, w0t[D1:], (((1,), (0,)), ((), ())),
                                preferred_element_type=jnp.float32))
    h0_ref[0] = h0

    row = jax.lax.broadcasted_iota(jnp.int32, (BN, 1), 0) + nb * BN
    m_row = (row < plens_ref[b]).astype(jnp.float32)
    s1 = jnp.sum(h0 * m_row, axis=0, keepdims=True)
    s2 = jnp.sum(h0 * h0 * m_row, axis=0, keepdims=True)
    riota = jax.lax.broadcasted_iota(jnp.int32, (8, C0), 0)
    contrib = (jnp.where(riota == 0, jnp.broadcast_to(s1, (8, C0)), 0.0)
               + jnp.where(riota == 1, jnp.broadcast_to(s2, (8, C0)), 0.0))

    @pl.when(jnp.logical_and(b == 0, nb == 0))
    def _():
        stats_ref[...] = jnp.zeros((8, C0), jnp.float32)

    stats_ref[...] += contrib


def _n_valid(plens_ref):
    nv = jnp.int32(0)
    for i in range(B):
        nv = nv + plens_ref[i]
    return jnp.maximum(nv.astype(jnp.float32), 1.0)


def _bn_mlp_body(plens_ref, h0_ref, stats_ref, g_ref, bias_ref, w1t_ref,
                 h1_ref, stats2_ref, *, cin, cout):
    b = pl.program_id(0)
    nb = pl.program_id(1)
    nv = _n_valid(plens_ref)
    stats = stats_ref[...]
    mean = stats[0:1, :] / nv
    var = stats[1:2, :] / nv - mean * mean
    scale = g_ref[...] * jax.lax.rsqrt(var + 1e-5)
    shift = bias_ref[...] - mean * scale
    xn = jnp.maximum(h0_ref[0] * scale + shift, 0.0)
    h1 = jax.lax.dot_general(xn, w1t_ref[...], (((1,), (0,)), ((), ())),
                             preferred_element_type=jnp.float32)
    h1_ref[0] = h1

    row = jax.lax.broadcasted_iota(jnp.int32, (BN, 1), 0) + nb * BN
    m_row = (row < plens_ref[b]).astype(jnp.float32)
    s1 = jnp.sum(h1 * m_row, axis=0, keepdims=True)
    s2 = jnp.sum(h1 * h1 * m_row, axis=0, keepdims=True)
    riota = jax.lax.broadcasted_iota(jnp.int32, (8, cout), 0)
    contrib = (jnp.where(riota == 0, jnp.broadcast_to(s1, (8, cout)), 0.0)
               + jnp.where(riota == 1, jnp.broadcast_to(s2, (8, cout)), 0.0))

    @pl.when(jnp.logical_and(b == 0, nb == 0))
    def _():
        stats2_ref[...] = jnp.zeros((8, cout), jnp.float32)

    stats2_ref[...] += contrib


def _bn_final_body(plens_ref, h1_ref, stats_ref, g_ref, bias_ref, out_ref, *,
                   cin):
    nv = _n_valid(plens_ref)
    stats = stats_ref[...]
    mean = stats[0:1, :] / nv
    var = stats[1:2, :] / nv - mean * mean
    scale = g_ref[...] * jax.lax.rsqrt(var + 1e-5)
    shift = bias_ref[...] - mean * scale
    out_ref[0] = jnp.maximum(h1_ref[0] * scale + shift, 0.0)


def kernel(xyz1, xyz2, points1, points2, point_lens, embedding_lens,
           point_mask, W0, g0, b0, W1, g1, b1):
    del point_mask
    xyz2t = xyz2.transpose(0, 2, 1)
    w0t = W0.T
    w1t = W1.T
    g0r, b0r = g0.reshape(1, C0), b0.reshape(1, C0)
    g1r, b1r = g1.reshape(1, C1), b1.reshape(1, C1)

    grid = (B, N // BN)
    smem = pl.BlockSpec(memory_space=pltpu.SMEM)

    h0, stats0 = pl.pallas_call(
        _knn_interp_body,
        grid=grid,
        in_specs=[
            smem, smem,
            pl.BlockSpec((1, BN, 3), lambda b, n: (b, n, 0)),
            pl.BlockSpec((1, 3, S), lambda b, n: (b, 0, 0)),
            pl.BlockSpec((1, BN, D1), lambda b, n: (b, n, 0)),
            pl.BlockSpec((1, S, D2), lambda b, n: (b, 0, 0)),
            pl.BlockSpec((D1 + D2, C0), lambda b, n: (0, 0)),
        ],
        out_specs=[
            pl.BlockSpec((1, BN, C0), lambda b, n: (b, n, 0)),
            pl.BlockSpec((8, C0), lambda b, n: (0, 0)),
        ],
        out_shape=[
            jax.ShapeDtypeStruct((B, N, C0), jnp.float32),
            jax.ShapeDtypeStruct((8, C0), jnp.float32),
        ],
    )(point_lens, embedding_lens, xyz1, xyz2t, points1, points2, w0t)

    h1, stats1 = pl.pallas_call(
        functools.partial(_bn_mlp_body, cin=C0, cout=C1),
        grid=grid,
        in_specs=[
            smem,
            pl.BlockSpec((1, BN, C0), lambda b, n: (b, n, 0)),
            pl.BlockSpec((8, C0), lambda b, n: (0, 0)),
            pl.BlockSpec((1, C0), lambda b, n: (0, 0)),
            pl.BlockSpec((1, C0), lambda b, n: (0, 0)),
            pl.BlockSpec((C0, C1), lambda b, n: (0, 0)),
        ],
        out_specs=[
            pl.BlockSpec((1, BN, C1), lambda b, n: (b, n, 0)),
            pl.BlockSpec((8, C1), lambda b, n: (0, 0)),
        ],
        out_shape=[
            jax.ShapeDtypeStruct((B, N, C1), jnp.float32),
            jax.ShapeDtypeStruct((8, C1), jnp.float32),
        ],
    )(point_lens, h0, stats0, g0r, b0r, w1t)

    out = pl.pallas_call(
        functools.partial(_bn_final_body, cin=C1),
        grid=grid,
        in_specs=[
            smem,
            pl.BlockSpec((1, BN, C1), lambda b, n: (b, n, 0)),
            pl.BlockSpec((8, C1), lambda b, n: (0, 0)),
            pl.BlockSpec((1, C1), lambda b, n: (0, 0)),
            pl.BlockSpec((1, C1), lambda b, n: (0, 0)),
        ],
        out_specs=pl.BlockSpec((1, BN, C1), lambda b, n: (b, n, 0)),
        out_shape=jax.ShapeDtypeStruct((B, N, C1), jnp.float32),
    )(point_lens, h1, stats1, g1r, b1r)

    return out


# speedup vs baseline: 30.5537x; 1.0589x over previous
import functools

import jax
import jax.numpy as jnp
from jax.experimental import pallas as pl
from jax.experimental.pallas import tpu as pltpu

B, N, S, D1, D2, K = 8, 4096, 1024, 128, 256, 5
C0, C1 = 256, 128
BN = 2048
INF = 3e38
BIG = 1e37
EPS = 1.1920928955078125e-07


def _knn_interp_body(plens_ref, elens_ref, xyz1_ref, xyz2t_ref, p1_ref, p2_ref,
                     w0t_ref, h0_ref, stats_ref):
    b = pl.program_id(0)
    nb = pl.program_id(1)

    a = xyz1_ref[0]
    bt = xyz2t_ref[0]
    an = jnp.sum(a * a, axis=1, keepdims=True)
    bn_sq = jnp.sum(bt * bt, axis=0, keepdims=True)
    col_s = jax.lax.broadcasted_iota(jnp.int32, (1, S), 1)
    elen = elens_ref[b]
    bn_m = jnp.where(col_s < elen, bn_sq, BIG)
    ab2 = jax.lax.dot_general(a * (-2.0), bt, (((1,), (0,)), ((), ())),
                              precision=jax.lax.Precision.HIGHEST,
                              preferred_element_type=jnp.float32)
    d0 = jnp.maximum(ab2 + an + bn_m, 0.0)

    d = d0
    wsum = jnp.zeros((BN, 1), jnp.float32)
    for _ in range(K):
        m = jnp.min(d, axis=1, keepdims=True)
        wsum = wsum + 1.0 / (m + EPS)
        d = jnp.where(d == m, INF, d)

    wsp = jnp.where(d == INF, 1.0 / (d0 + EPS), 0.0)
    interp = jax.lax.dot_general(wsp, p2_ref[0], (((1,), (0,)), ((), ())),
                                 preferred_element_type=jnp.float32)
    interp = interp * (1.0 / wsum)
    w0t = w0t_ref[...]
    h0 = (jax.lax.dot_general(p1_ref[0], w0t[:D1], (((1,), (0,)), ((), ())),
                              preferred_element_type=jnp.float32)
          + jax.lax.dot_general(interp, w0t[D1:], (((1,), (0,)), ((), ())),
                                preferred_element_type=jnp.float32))
    h0_ref[0] = h0

    row = jax.lax.broadcasted_iota(jnp.int32, (BN, 1), 0) + nb * BN
    m_row = (row < plens_ref[b]).astype(jnp.float32)
    s1 = jnp.sum(h0 * m_row, axis=0, keepdims=True)
    s2 = jnp.sum(h0 * h0 * m_row, axis=0, keepdims=True)
    riota = jax.lax.broadcasted_iota(jnp.int32, (8, C0), 0)
    contrib = (jnp.where(riota == 0, jnp.broadcast_to(s1, (8, C0)), 0.0)
               + jnp.where(riota == 1, jnp.broadcast_to(s2, (8, C0)), 0.0))

    @pl.when(jnp.logical_and(b == 0, nb == 0))
    def _():
        stats_ref[...] = jnp.zeros((8, C0), jnp.float32)

    stats_ref[...] += contrib


def _n_valid(plens_ref):
    nv = jnp.int32(0)
    for i in range(B):
        nv = nv + plens_ref[i]
    return jnp.maximum(nv.astype(jnp.float32), 1.0)


def _bn_mlp_body(plens_ref, h0_ref, stats_ref, g_ref, bias_ref, w1t_ref,
                 h1_ref, stats2_ref, *, cin, cout):
    b = pl.program_id(0)
    nb = pl.program_id(1)
    nv = _n_valid(plens_ref)
    stats = stats_ref[...]
    mean = stats[0:1, :] / nv
    var = stats[1:2, :] / nv - mean * mean
    scale = g_ref[...] * jax.lax.rsqrt(var + 1e-5)
    shift = bias_ref[...] - mean * scale
    xn = jnp.maximum(h0_ref[0] * scale + shift, 0.0)
    h1 = jax.lax.dot_general(xn, w1t_ref[...], (((1,), (0,)), ((), ())),
                             preferred_element_type=jnp.float32)
    h1_ref[0] = h1

    row = jax.lax.broadcasted_iota(jnp.int32, (BN, 1), 0) + nb * BN
    m_row = (row < plens_ref[b]).astype(jnp.float32)
    s1 = jnp.sum(h1 * m_row, axis=0, keepdims=True)
    s2 = jnp.sum(h1 * h1 * m_row, axis=0, keepdims=True)
    riota = jax.lax.broadcasted_iota(jnp.int32, (8, cout), 0)
    contrib = (jnp.where(riota == 0, jnp.broadcast_to(s1, (8, cout)), 0.0)
               + jnp.where(riota == 1, jnp.broadcast_to(s2, (8, cout)), 0.0))

    @pl.when(jnp.logical_and(b == 0, nb == 0))
    def _():
        stats2_ref[...] = jnp.zeros((8, cout), jnp.float32)

    stats2_ref[...] += contrib


def _bn_final_body(plens_ref, h1_ref, stats_ref, g_ref, bias_ref, out_ref, *,
                   cin):
    nv = _n_valid(plens_ref)
    stats = stats_ref[...]
    mean = stats[0:1, :] / nv
    var = stats[1:2, :] / nv - mean * mean
    scale = g_ref[...] * jax.lax.rsqrt(var + 1e-5)
    shift = bias_ref[...] - mean * scale
    out_ref[0] = jnp.maximum(h1_ref[0] * scale + shift, 0.0)


def kernel(xyz1, xyz2, points1, points2, point_lens, embedding_lens,
           point_mask, W0, g0, b0, W1, g1, b1):
    del point_mask
    xyz2t = xyz2.transpose(0, 2, 1)
    w0t = W0.T
    w1t = W1.T
    g0r, b0r = g0.reshape(1, C0), b0.reshape(1, C0)
    g1r, b1r = g1.reshape(1, C1), b1.reshape(1, C1)

    grid = (B, N // BN)
    smem = pl.BlockSpec(memory_space=pltpu.SMEM)

    h0, stats0 = pl.pallas_call(
        _knn_interp_body,
        grid=grid,
        in_specs=[
            smem, smem,
            pl.BlockSpec((1, BN, 3), lambda b, n: (b, n, 0)),
            pl.BlockSpec((1, 3, S), lambda b, n: (b, 0, 0)),
            pl.BlockSpec((1, BN, D1), lambda b, n: (b, n, 0)),
            pl.BlockSpec((1, S, D2), lambda b, n: (b, 0, 0)),
            pl.BlockSpec((D1 + D2, C0), lambda b, n: (0, 0)),
        ],
        out_specs=[
            pl.BlockSpec((1, BN, C0), lambda b, n: (b, n, 0)),
            pl.BlockSpec((8, C0), lambda b, n: (0, 0)),
        ],
        out_shape=[
            jax.ShapeDtypeStruct((B, N, C0), jnp.float32),
            jax.ShapeDtypeStruct((8, C0), jnp.float32),
        ],
    )(point_lens, embedding_lens, xyz1, xyz2t, points1, points2, w0t)

    h1, stats1 = pl.pallas_call(
        functools.partial(_bn_mlp_body, cin=C0, cout=C1),
        grid=grid,
        in_specs=[
            smem,
            pl.BlockSpec((1, BN, C0), lambda b, n: (b, n, 0)),
            pl.BlockSpec((8, C0), lambda b, n: (0, 0)),
            pl.BlockSpec((1, C0), lambda b, n: (0, 0)),
            pl.BlockSpec((1, C0), lambda b, n: (0, 0)),
            pl.BlockSpec((C0, C1), lambda b, n: (0, 0)),
        ],
        out_specs=[
            pl.BlockSpec((1, BN, C1), lambda b, n: (b, n, 0)),
            pl.BlockSpec((8, C1), lambda b, n: (0, 0)),
        ],
        out_shape=[
            jax.ShapeDtypeStruct((B, N, C1), jnp.float32),
            jax.ShapeDtypeStruct((8, C1), jnp.float32),
        ],
    )(point_lens, h0, stats0, g0r, b0r, w1t)

    out = pl.pallas_call(
        functools.partial(_bn_final_body, cin=C1),
        grid=grid,
        in_specs=[
            smem,
            pl.BlockSpec((1, BN, C1), lambda b, n: (b, n, 0)),
            pl.BlockSpec((8, C1), lambda b, n: (0, 0)),
            pl.BlockSpec((1, C1), lambda b, n: (0, 0)),
            pl.BlockSpec((1, C1), lambda b, n: (0, 0)),
        ],
        out_specs=pl.BlockSpec((1, BN, C1), lambda b, n: (b, n, 0)),
        out_shape=jax.ShapeDtypeStruct((B, N, C1), jnp.float32),
    )(point_lens, h1, stats1, g1r, b1r)

    return out


# speedup vs baseline: 31.8531x; 1.0425x over previous
import functools

import jax
import jax.numpy as jnp
from jax.experimental import pallas as pl
from jax.experimental.pallas import tpu as pltpu

B, N, S, D1, D2, K = 8, 4096, 1024, 128, 256, 5
C0, C1 = 256, 128
BN = 4096
INF = 3e38
BIG = 1e37
EPS = 1.1920928955078125e-07


def _knn_interp_body(plens_ref, elens_ref, xyz1_ref, xyz2t_ref, p1_ref, p2_ref,
                     w0t_ref, h0_ref, stats_ref):
    b = pl.program_id(0)
    nb = pl.program_id(1)

    a = xyz1_ref[0]
    bt = xyz2t_ref[0]
    an = jnp.sum(a * a, axis=1, keepdims=True)
    bn_sq = jnp.sum(bt * bt, axis=0, keepdims=True)
    col_s = jax.lax.broadcasted_iota(jnp.int32, (1, S), 1)
    elen = elens_ref[b]
    bn_m = jnp.where(col_s < elen, bn_sq, BIG)
    ab2 = jax.lax.dot_general(a * (-2.0), bt, (((1,), (0,)), ((), ())),
                              precision=jax.lax.Precision.HIGHEST,
                              preferred_element_type=jnp.float32)
    d0 = jnp.maximum(ab2 + an + bn_m, 0.0)

    d = d0
    wsum = jnp.zeros((BN, 1), jnp.float32)
    for _ in range(K):
        m = jnp.min(d, axis=1, keepdims=True)
        wsum = wsum + 1.0 / (m + EPS)
        d = jnp.where(d == m, INF, d)

    wsp = jnp.where(d == INF, 1.0 / (d0 + EPS), 0.0)
    interp = jax.lax.dot_general(wsp, p2_ref[0], (((1,), (0,)), ((), ())),
                                 preferred_element_type=jnp.float32)
    interp = interp * (1.0 / wsum)
    w0t = w0t_ref[...]
    h0 = (jax.lax.dot_general(p1_ref[0], w0t[:D1], (((1,), (0,)), ((), ())),
                              preferred_element_type=jnp.float32)
          + jax.lax.dot_general(interp, w0t[D1:], (((1,), (0,)), ((), ())),
                                preferred_element_type=jnp.float32))
    h0_ref[0] = h0

    row = jax.lax.broadcasted_iota(jnp.int32, (BN, 1), 0) + nb * BN
    m_row = (row < plens_ref[b]).astype(jnp.float32)
    s1 = jnp.sum(h0 * m_row, axis=0, keepdims=True)
    s2 = jnp.sum(h0 * h0 * m_row, axis=0, keepdims=True)
    riota = jax.lax.broadcasted_iota(jnp.int32, (8, C0), 0)
    contrib = (jnp.where(riota == 0, jnp.broadcast_to(s1, (8, C0)), 0.0)
               + jnp.where(riota == 1, jnp.broadcast_to(s2, (8, C0)), 0.0))

    @pl.when(jnp.logical_and(b == 0, nb == 0))
    def _():
        stats_ref[...] = jnp.zeros((8, C0), jnp.float32)

    stats_ref[...] += contrib


def _n_valid(plens_ref):
    nv = jnp.int32(0)
    for i in range(B):
        nv = nv + plens_ref[i]
    return jnp.maximum(nv.astype(jnp.float32), 1.0)


def _bn_mlp_body(plens_ref, h0_ref, stats_ref, g_ref, bias_ref, w1t_ref,
                 h1_ref, stats2_ref, *, cin, cout):
    b = pl.program_id(0)
    nb = pl.program_id(1)
    nv = _n_valid(plens_ref)
    stats = stats_ref[...]
    mean = stats[0:1, :] / nv
    var = stats[1:2, :] / nv - mean * mean
    scale = g_ref[...] * jax.lax.rsqrt(var + 1e-5)
    shift = bias_ref[...] - mean * scale
    xn = jnp.maximum(h0_ref[0] * scale + shift, 0.0)
    h1 = jax.lax.dot_general(xn, w1t_ref[...], (((1,), (0,)), ((), ())),
                             preferred_element_type=jnp.float32)
    h1_ref[0] = h1

    row = jax.lax.broadcasted_iota(jnp.int32, (BN, 1), 0) + nb * BN
    m_row = (row < plens_ref[b]).astype(jnp.float32)
    s1 = jnp.sum(h1 * m_row, axis=0, keepdims=True)
    s2 = jnp.sum(h1 * h1 * m_row, axis=0, keepdims=True)
    riota = jax.lax.broadcasted_iota(jnp.int32, (8, cout), 0)
    contrib = (jnp.where(riota == 0, jnp.broadcast_to(s1, (8, cout)), 0.0)
               + jnp.where(riota == 1, jnp.broadcast_to(s2, (8, cout)), 0.0))

    @pl.when(jnp.logical_and(b == 0, nb == 0))
    def _():
        stats2_ref[...] = jnp.zeros((8, cout), jnp.float32)

    stats2_ref[...] += contrib


def _bn_final_body(plens_ref, h1_ref, stats_ref, g_ref, bias_ref, out_ref, *,
                   cin):
    nv = _n_valid(plens_ref)
    stats = stats_ref[...]
    mean = stats[0:1, :] / nv
    var = stats[1:2, :] / nv - mean * mean
    scale = g_ref[...] * jax.lax.rsqrt(var + 1e-5)
    shift = bias_ref[...] - mean * scale
    out_ref[0] = jnp.maximum(h1_ref[0] * scale + shift, 0.0)


def kernel(xyz1, xyz2, points1, points2, point_lens, embedding_lens,
           point_mask, W0, g0, b0, W1, g1, b1):
    del point_mask
    xyz2t = xyz2.transpose(0, 2, 1)
    w0t = W0.T
    w1t = W1.T
    g0r, b0r = g0.reshape(1, C0), b0.reshape(1, C0)
    g1r, b1r = g1.reshape(1, C1), b1.reshape(1, C1)

    grid = (B, N // BN)
    smem = pl.BlockSpec(memory_space=pltpu.SMEM)

    h0, stats0 = pl.pallas_call(
        _knn_interp_body,
        grid=grid,
        in_specs=[
            smem, smem,
            pl.BlockSpec((1, BN, 3), lambda b, n: (b, n, 0)),
            pl.BlockSpec((1, 3, S), lambda b, n: (b, 0, 0)),
            pl.BlockSpec((1, BN, D1), lambda b, n: (b, n, 0)),
            pl.BlockSpec((1, S, D2), lambda b, n: (b, 0, 0)),
            pl.BlockSpec((D1 + D2, C0), lambda b, n: (0, 0)),
        ],
        out_specs=[
            pl.BlockSpec((1, BN, C0), lambda b, n: (b, n, 0)),
            pl.BlockSpec((8, C0), lambda b, n: (0, 0)),
        ],
        out_shape=[
            jax.ShapeDtypeStruct((B, N, C0), jnp.float32),
            jax.ShapeDtypeStruct((8, C0), jnp.float32),
        ],
    )(point_lens, embedding_lens, xyz1, xyz2t, points1, points2, w0t)

    h1, stats1 = pl.pallas_call(
        functools.partial(_bn_mlp_body, cin=C0, cout=C1),
        grid=grid,
        in_specs=[
            smem,
            pl.BlockSpec((1, BN, C0), lambda b, n: (b, n, 0)),
            pl.BlockSpec((8, C0), lambda b, n: (0, 0)),
            pl.BlockSpec((1, C0), lambda b, n: (0, 0)),
            pl.BlockSpec((1, C0), lambda b, n: (0, 0)),
            pl.BlockSpec((C0, C1), lambda b, n: (0, 0)),
        ],
        out_specs=[
            pl.BlockSpec((1, BN, C1), lambda b, n: (b, n, 0)),
            pl.BlockSpec((8, C1), lambda b, n: (0, 0)),
        ],
        out_shape=[
            jax.ShapeDtypeStruct((B, N, C1), jnp.float32),
            jax.ShapeDtypeStruct((8, C1), jnp.float32),
        ],
    )(point_lens, h0, stats0, g0r, b0r, w1t)

    out = pl.pallas_call(
        functools.partial(_bn_final_body, cin=C1),
        grid=grid,
        in_specs=[
            smem,
            pl.BlockSpec((1, BN, C1), lambda b, n: (b, n, 0)),
            pl.BlockSpec((8, C1), lambda b, n: (0, 0)),
            pl.BlockSpec((1, C1), lambda b, n: (0, 0)),
            pl.BlockSpec((1, C1), lambda b, n: (0, 0)),
        ],
        out_specs=pl.BlockSpec((1, BN, C1), lambda b, n: (b, n, 0)),
        out_shape=jax.ShapeDtypeStruct((B, N, C1), jnp.float32),
    )(point_lens, h1, stats1, g1r, b1r)

    return out
